# Initial kernel scaffold; baseline (speedup 1.0000x reference)
import jax
import jax.numpy as jnp
from jax.experimental import pallas as pl


def kernel(x, edge_index, drop, W1, b1, W2, b2):
    raise NotImplementedError("write your pallas kernel here")


# speedup vs baseline: 10.3387x; 10.3387x over previous
import functools

import jax
import jax.numpy as jnp
from jax import lax
from jax.experimental import pallas as pl
from jax.experimental.pallas import tpu as pltpu
from jax.experimental.pallas import tpu_sc as plsc

N = 10000
D = 128
E = 320000
NC = 2
NS = 16
NW = NC * NS
CHUNK = 128
CPW = 79
E_PAD = NW * CPW * CHUNK
NPAD = 10112
RPT = NPAD // NS

BLK = 400
GRID = N // BLK

_mesh = plsc.VectorSubcoreMesh(core_axis_name="c", subcore_axis_name="s")


DW = 16


@functools.partial(
    pl.kernel,
    mesh=_mesh,
    out_type=(jax.ShapeDtypeStruct((NPAD, DW), jnp.float32),
              jax.ShapeDtypeStruct((NPAD, DW), jnp.float32)),
    scratch_types=[
        pltpu.VMEM((CHUNK,), jnp.int32),
        pltpu.VMEM((CHUNK, DW), jnp.float32),
        pltpu.VMEM((CHUNK, DW), jnp.float32),
        pltpu.VMEM_SHARED((NPAD, DW), jnp.float32),
    ],
)
def _deg_sc(dst_hbm, deg0_hbm, deg1_hbm, dst_v, ones_b, zb16, dacc):
    c = lax.axis_index("c")
    s = lax.axis_index("s")
    one = jnp.ones((DW,), jnp.float32)
    zv = jnp.zeros((DW,), jnp.float32)

    def fill(i, carry):
        ones_b[i, :] = one
        zb16[i, :] = zv
        return carry

    lax.fori_loop(0, CHUNK, fill, None)
    base = s * RPT
    for k in range(RPT // CHUNK):
        pltpu.sync_copy(zb16, dacc.at[pl.ds(base + k * CHUNK, CHUNK)])
    rem = RPT % CHUNK
    if rem:
        pltpu.sync_copy(zb16.at[pl.ds(0, rem)],
                        dacc.at[pl.ds(base + RPT - rem, rem)])
    plsc.subcore_barrier()

    wid = c * NS + s

    def chunk_body(j, carry):
        off = (wid * CPW + j) * CHUNK
        pltpu.sync_copy(dst_hbm.at[pl.ds(off, CHUNK)], dst_v)
        pltpu.sync_copy(ones_b, dacc.at[dst_v], add=True)
        return carry

    lax.fori_loop(0, CPW, chunk_body, None)
    plsc.subcore_barrier()

    @pl.when(c == 0)
    def _():
        pltpu.sync_copy(dacc.at[pl.ds(base, RPT)],
                        deg0_hbm.at[pl.ds(base, RPT)])

    @pl.when(c == 1)
    def _():
        pltpu.sync_copy(dacc.at[pl.ds(base, RPT)],
                        deg1_hbm.at[pl.ds(base, RPT)])


@functools.partial(
    pl.kernel,
    mesh=_mesh,
    out_type=(jax.ShapeDtypeStruct((NPAD, D), jnp.float32),
              jax.ShapeDtypeStruct((NPAD, D), jnp.float32)),
    scratch_types=[
        pltpu.VMEM((CHUNK,), jnp.int32),
        pltpu.VMEM((CHUNK,), jnp.int32),
        pltpu.VMEM((CHUNK, D), jnp.float32),
        pltpu.VMEM((CHUNK, D), jnp.float32),
        pltpu.VMEM_SHARED((NPAD, D), jnp.float32),
        pltpu.SemaphoreType.DMA,
    ],
)
def _prop_sc(xs_hbm, src_hbm, dst_hbm, out0_hbm, out1_hbm,
             src_v, dst_v, rows, zbuf, accum, sem):
    c = lax.axis_index("c")
    s = lax.axis_index("s")
    zv = jnp.zeros((16,), jnp.float32)

    def zb(i, carry):
        zbuf[i >> 3, pl.ds((i & 7) * 16, 16)] = zv
        return carry

    lax.fori_loop(0, CHUNK * (D // 16), zb, None)
    base = s * RPT
    for k in range(RPT // CHUNK):
        pltpu.sync_copy(zbuf, accum.at[pl.ds(base + k * CHUNK, CHUNK)])
    rem = RPT % CHUNK
    if rem:
        pltpu.sync_copy(zbuf.at[pl.ds(0, rem)],
                        accum.at[pl.ds(base + RPT - rem, rem)])
    plsc.subcore_barrier()

    wid = c * NS + s

    def chunk_body(j, carry):
        off = (wid * CPW + j) * CHUNK
        pltpu.sync_copy(src_hbm.at[pl.ds(off, CHUNK)], src_v)
        pltpu.sync_copy(dst_hbm.at[pl.ds(off, CHUNK)], dst_v)
        pltpu.async_copy(xs_hbm.at[src_v], rows, sem).wait()
        pltpu.sync_copy(rows, accum.at[dst_v], add=True)
        return carry

    lax.fori_loop(0, CPW, chunk_body, None)
    plsc.subcore_barrier()

    @pl.when(c == 0)
    def _():
        pltpu.sync_copy(accum.at[pl.ds(base, RPT)],
                        out0_hbm.at[pl.ds(base, RPT)])

    @pl.when(c == 1)
    def _():
        pltpu.sync_copy(accum.at[pl.ds(base, RPT)],
                        out1_hbm.at[pl.ds(base, RPT)])


def _dis_from(d0_ref, d1_ref):
    deg = d0_ref[...] + d1_ref[...]
    return lax.rsqrt(deg[:, :1] + 1.0)


def _tc1_body(x_ref, w_ref, d0_ref, d1_ref, o_ref):
    dis = _dis_from(d0_ref, d1_ref)
    y = jnp.dot(x_ref[...], w_ref[...], preferred_element_type=jnp.float32)
    o_ref[...] = y * dis


def _tc1(x, W1, d0, d1):
    return pl.pallas_call(
        _tc1_body,
        grid=(GRID,),
        in_specs=[
            pl.BlockSpec((BLK, D), lambda j: (j, 0)),
            pl.BlockSpec((D, D), lambda j: (0, 0)),
            pl.BlockSpec((BLK, DW), lambda j: (j, 0)),
            pl.BlockSpec((BLK, DW), lambda j: (j, 0)),
        ],
        out_specs=pl.BlockSpec((BLK, D), lambda j: (j, 0)),
        out_shape=jax.ShapeDtypeStruct((N, D), jnp.float32),
    )(x, W1, d0, d1)


def _tc2_body(p0_ref, p1_ref, xs_ref, d0_ref, d1_ref, b_ref, w_ref, o_ref):
    dis = _dis_from(d0_ref, d1_ref)
    h = (p0_ref[...] + p1_ref[...] + xs_ref[...]) * dis + b_ref[...]
    h = jnp.maximum(h, 0.0)
    o_ref[...] = jnp.dot(h, w_ref[...],
                         preferred_element_type=jnp.float32) * dis


def _tc2(p0, p1, xs1, d0, d1, b1r, W2):
    return pl.pallas_call(
        _tc2_body,
        grid=(GRID,),
        in_specs=[
            pl.BlockSpec((BLK, D), lambda j: (j, 0)),
            pl.BlockSpec((BLK, D), lambda j: (j, 0)),
            pl.BlockSpec((BLK, D), lambda j: (j, 0)),
            pl.BlockSpec((BLK, DW), lambda j: (j, 0)),
            pl.BlockSpec((BLK, DW), lambda j: (j, 0)),
            pl.BlockSpec((1, D), lambda j: (0, 0)),
            pl.BlockSpec((D, D), lambda j: (0, 0)),
        ],
        out_specs=pl.BlockSpec((BLK, D), lambda j: (j, 0)),
        out_shape=jax.ShapeDtypeStruct((N, D), jnp.float32),
    )(p0, p1, xs1, d0, d1, b1r, W2)


def _tc3_body(p0_ref, p1_ref, xs_ref, d0_ref, d1_ref, b_ref, o_ref):
    dis = _dis_from(d0_ref, d1_ref)
    h = (p0_ref[...] + p1_ref[...] + xs_ref[...]) * dis + b_ref[...]
    nrm = jnp.sqrt(jnp.sum(h * h, axis=1, keepdims=True))
    h = h / jnp.maximum(nrm, 1e-12)
    m = jnp.max(h, axis=1, keepdims=True)
    e = h - m
    o_ref[...] = e - jnp.log(jnp.sum(jnp.exp(e), axis=1, keepdims=True))


def _tc3(p0, p1, xs2, d0, d1, b2r):
    return pl.pallas_call(
        _tc3_body,
        grid=(GRID,),
        in_specs=[
            pl.BlockSpec((BLK, D), lambda j: (j, 0)),
            pl.BlockSpec((BLK, D), lambda j: (j, 0)),
            pl.BlockSpec((BLK, D), lambda j: (j, 0)),
            pl.BlockSpec((BLK, DW), lambda j: (j, 0)),
            pl.BlockSpec((BLK, DW), lambda j: (j, 0)),
            pl.BlockSpec((1, D), lambda j: (0, 0)),
        ],
        out_specs=pl.BlockSpec((BLK, D), lambda j: (j, 0)),
        out_shape=jax.ShapeDtypeStruct((N, D), jnp.float32),
    )(p0, p1, xs2, d0, d1, b2r)


def kernel(x, edge_index, drop, W1, b1, W2, b2):
    src = edge_index[0].astype(jnp.int32)
    dst = edge_index[1].astype(jnp.int32)
    padn = E_PAD - E
    src_p = jnp.concatenate([src, jnp.zeros((padn,), jnp.int32)])
    dst_p = jnp.concatenate([dst, jnp.full((padn,), N, jnp.int32)])

    d0, d1 = _deg_sc(dst_p)
    b1r = b1.reshape(1, D)
    b2r = b2.reshape(1, D)

    xs1 = _tc1(x, W1, d0, d1)
    p10, p11 = _prop_sc(xs1, src_p, dst_p)
    xs2 = _tc2(p10, p11, xs1, d0, d1, b1r, W2)
    p20, p21 = _prop_sc(xs2, src_p, dst_p)
    return _tc3(p20, p21, xs2, d0, d1, b2r)
